# baseline (speedup 1.0000x reference)
import functools

import jax
import jax.numpy as jnp
from jax import lax
from jax.experimental import pallas as pl
from jax.experimental.pallas import tpu as pltpu
from jax.experimental.pallas import tpu_sc as plsc

SEQ = 4
N = 10000
D = 128
E = 320000
B = 100
H = 128

NCORES = 2
NSUB = 16
NW = NCORES * NSUB
K = 128
EPT = 10240
C = EPT // K
RPT = 624
REXTRA = N - NSUB * RPT
ZR = 78


def _mm_body(x_ref, w_ref, o_ref):
    o_ref[...] = jnp.dot(x_ref[...], w_ref[...],
                         preferred_element_type=jnp.float32)


def _support_matmul(xs_flat, w):
    M = SEQ * N
    BM = 1000
    return pl.pallas_call(
        _mm_body,
        grid=(M // BM,),
        in_specs=[
            pl.BlockSpec((BM, D), lambda i: (i, 0)),
            pl.BlockSpec((D, D), lambda i: (0, 0)),
        ],
        out_specs=pl.BlockSpec((BM, D), lambda i: (i, 0)),
        out_shape=jax.ShapeDtypeStruct((M, D), jnp.float32),
    )(xs_flat, w)


def _bcast_lane(v16, lane):
    idx = jnp.full((16, 1), lane, dtype=jnp.int32)
    return lax.gather(
        v16, idx,
        lax.GatherDimensionNumbers(offset_dims=(), collapsed_slice_dims=(0,),
                                   start_index_map=(0,)),
        (1,), mode=lax.GatherScatterMode.PROMISE_IN_BOUNDS)


CQ = 40
NQ = C // CQ


def _gcn_aggregate(support_flat, cols_p, rows_p, adj_p):
    mesh = plsc.VectorSubcoreMesh(core_axis_name="c", subcore_axis_name="s")

    @functools.partial(
        pl.kernel,
        out_type=jax.ShapeDtypeStruct((NCORES, SEQ, N, D), jnp.float32),
        mesh=mesh,
        scratch_types=[
            pltpu.VMEM((CQ, K), jnp.int32),
            pltpu.VMEM((CQ, K), jnp.int32),
            pltpu.VMEM((CQ, K), jnp.float32),
            pltpu.VMEM((K, D), jnp.float32),
            pltpu.VMEM((K, D), jnp.float32),
            pltpu.VMEM_SHARED((N, D), jnp.float32),
            pltpu.SemaphoreType.DMA,
            pltpu.SemaphoreType.DMA,
            pltpu.SemaphoreType.DMA,
            pltpu.SemaphoreType.DMA,
        ],
    )
    def k(sup_hbm, cols_hbm, rows_hbm, adj_hbm, out_hbm,
          cols_all, rows_all, adj_all, buf0, buf1, acc,
          semg0, semg1, sems0, sems1):
        cid = lax.axis_index("c")
        sid = lax.axis_index("s")
        w = cid * NSUB + sid
        base = sid * RPT
        last = sid == NSUB - 1

        def zero_acc():
            z16 = jnp.zeros((16,), jnp.float32)
            for v in range(D // 16):
                def zrow(r, _, v=v):
                    buf0[r, pl.ds(v * 16, 16)] = z16
                    return _
                lax.fori_loop(0, ZR, zrow, None)
            for z in range(RPT // ZR):
                pltpu.sync_copy(buf0.at[pl.ds(0, ZR)],
                                acc.at[pl.ds(base + z * ZR, ZR)])

            @pl.when(last)
            def _():
                pltpu.sync_copy(buf0.at[pl.ds(0, REXTRA)],
                                acc.at[pl.ds(NSUB * RPT, REXTRA)])

        def gath(m, buf, sem):
            return pltpu.make_async_copy(sup_hbm.at[cols_all.at[m]], buf, sem)

        def scal(m, buf, sem):
            return pltpu.async_copy(buf, acc.at[rows_all.at[m]], sem,
                                    add=True)

        def scale(m, buf):
            def sg(g, _):
                a16 = adj_all[m, pl.ds(g * 16, 16)]
                for e in range(16):
                    jj = g * 16 + e
                    ae = _bcast_lane(a16, e)
                    for v in range(D // 16):
                        sl = pl.ds(v * 16, 16)
                        buf[jj, sl] = buf[jj, sl] * ae
                return _
            lax.fori_loop(0, K // 16, sg, None)

        zero_acc()
        plsc.subcore_barrier()

        for s in range(SEQ):
            for q in range(NQ):
                rbase = (s * NW + w) * C + q * CQ
                pltpu.sync_copy(cols_hbm.at[pl.ds(rbase, CQ)], cols_all)
                pltpu.sync_copy(rows_hbm.at[pl.ds(rbase, CQ)], rows_all)
                pltpu.sync_copy(adj_hbm.at[pl.ds(rbase, CQ)], adj_all)

                gath(0, buf0, semg0).start()
                gath(1, buf1, semg1).start()

                def body(j, _):
                    m0 = 2 * j
                    gath(m0, buf0, semg0).wait()
                    if False:
                        scale(m0, buf0)
                    sc0 = scal(m0, buf0, sems0)

                    gath(m0 + 1, buf1, semg1).wait()
                    if False:
                        scale(m0 + 1, buf1)
                    sc1 = scal(m0 + 1, buf1, sems1)

                    @pl.when(j < CQ // 2 - 1)
                    def _():
                        sc0.wait()
                        gath(m0 + 2, buf0, semg0).start()
                        sc1.wait()
                        gath(m0 + 3, buf1, semg1).start()

                    @pl.when(j == CQ // 2 - 1)
                    def _():
                        sc0.wait()
                        sc1.wait()
                    return _
                lax.fori_loop(0, CQ // 2, body, None)
            plsc.subcore_barrier()
            pltpu.sync_copy(acc.at[pl.ds(base, RPT)],
                            out_hbm.at[cid, s, pl.ds(base, RPT)])

            @pl.when(last)
            def _():
                pltpu.sync_copy(acc.at[pl.ds(NSUB * RPT, REXTRA)],
                                out_hbm.at[cid, s, pl.ds(NSUB * RPT, REXTRA)])

            if s < SEQ - 1:
                zero_acc()
            plsc.subcore_barrier()

    return k(support_flat, cols_p, rows_p, adj_p)


BT = 10


def _lstm_body(p0_ref, p1_ref, gb_ref, wih_ref, whh_ref, bias_ref,
               h0_ref, c0_ref, out_ref, h_scr, c_scr):
    t = pl.program_id(0)

    @pl.when(t == 0)
    def _():
        h_scr[...] = h0_ref[...]
        c_scr[...] = c0_ref[...]

    wih = wih_ref[...]
    whh = whh_ref[...]
    bias = bias_ref[...]
    gbv = gb_ref[...]

    def step(i, hc):
        h, c = hc
        rel = jnp.maximum(p0_ref[i] + p1_ref[i] + gbv, 0.0)
        gates = (jnp.dot(rel, wih, preferred_element_type=jnp.float32)
                 + jnp.dot(h, whh, preferred_element_type=jnp.float32)
                 + bias)
        i_g = jax.nn.sigmoid(gates[:, :H])
        f_g = jax.nn.sigmoid(gates[:, H:2 * H])
        g_g = jnp.tanh(gates[:, 2 * H:3 * H])
        o_g = jax.nn.sigmoid(gates[:, 3 * H:])
        c2 = f_g * c + i_g * g_g
        h2 = o_g * jnp.tanh(c2)
        out_ref[i] = h2
        return (h2, c2)

    h, c = lax.fori_loop(0, BT, step, (h_scr[...], c_scr[...]))
    h_scr[...] = h
    c_scr[...] = c


def _lstm(p0, p1, gb, wih_t, whh_t, bias, h0, c0):
    T = SEQ * N // B
    return pl.pallas_call(
        _lstm_body,
        grid=(T // BT,),
        in_specs=[
            pl.BlockSpec((BT, B, H), lambda i: (i, 0, 0)),
            pl.BlockSpec((BT, B, H), lambda i: (i, 0, 0)),
            pl.BlockSpec((1, H), lambda i: (0, 0)),
            pl.BlockSpec((D, 4 * H), lambda i: (0, 0)),
            pl.BlockSpec((H, 4 * H), lambda i: (0, 0)),
            pl.BlockSpec((1, 4 * H), lambda i: (0, 0)),
            pl.BlockSpec((B, H), lambda i: (0, 0)),
            pl.BlockSpec((B, H), lambda i: (0, 0)),
        ],
        out_specs=pl.BlockSpec((BT, B, H), lambda i: (i, 0, 0)),
        out_shape=jax.ShapeDtypeStruct((T, B, H), jnp.float32),
        scratch_shapes=[
            pltpu.VMEM((B, H), jnp.float32),
            pltpu.VMEM((B, H), jnp.float32),
        ],
    )(p0, p1, gb, wih_t, whh_t, bias, h0, c0)


def kernel(xs, edge_index, adj_values, gcn_w, gcn_b, W_ih, W_hh,
           b_ih, b_hh, h0, c0):
    xs_flat = xs.reshape(SEQ * N, D)
    support = _support_matmul(xs_flat, gcn_w)

    pad = NW * EPT - E
    cols = edge_index[:, 1, :] + (jnp.arange(SEQ, dtype=jnp.int32) * N)[:, None]
    rows = edge_index[:, 0, :]
    cols_p = jnp.concatenate(
        [cols, jnp.zeros((SEQ, pad), jnp.int32)], axis=1).reshape(-1, K)
    rows_p = jnp.concatenate(
        [rows, jnp.zeros((SEQ, pad), jnp.int32)], axis=1).reshape(-1, K)
    adj_p = jnp.concatenate(
        [adj_values, jnp.zeros((SEQ, pad), jnp.float32)], axis=1).reshape(-1, K)

    parts = _gcn_aggregate(support, cols_p, rows_p, adj_p)
    p0 = parts[0].reshape(SEQ * N // B, B, H)
    p1 = parts[1].reshape(SEQ * N // B, B, H)

    gb = gcn_b.reshape(1, H)
    wih_t = W_ih.T
    whh_t = W_hh.T
    bias = (b_ih + b_hh).reshape(1, 4 * H)

    return _lstm(p0, p1, gb, wih_t, whh_t, bias, h0[0], c0[0])


# speedup vs baseline: 2.4651x; 2.4651x over previous
import functools

import jax
import jax.numpy as jnp
from jax import lax
from jax.experimental import pallas as pl
from jax.experimental.pallas import tpu as pltpu
from jax.experimental.pallas import tpu_sc as plsc

SEQ = 4
N = 10000
D = 128
E = 320000
B = 100
H = 128

NCORES = 2
NSUB = 16
NW = NCORES * NSUB
K = 128
EPT = 10240
C = EPT // K
RPT = 624
REXTRA = N - NSUB * RPT
ZR = 78


def _mm_body(x_ref, w_ref, o_ref):
    o_ref[...] = jnp.dot(x_ref[...], w_ref[...],
                         preferred_element_type=jnp.float32)


def _support_matmul(xs_flat, w):
    M = SEQ * N
    BM = 1000
    return pl.pallas_call(
        _mm_body,
        grid=(M // BM,),
        in_specs=[
            pl.BlockSpec((BM, D), lambda i: (i, 0)),
            pl.BlockSpec((D, D), lambda i: (0, 0)),
        ],
        out_specs=pl.BlockSpec((BM, D), lambda i: (i, 0)),
        out_shape=jax.ShapeDtypeStruct((M, D), jnp.float32),
    )(xs_flat, w)


def _bcast_lane(v16, lane):
    idx = jnp.full((16, 1), lane, dtype=jnp.int32)
    return lax.gather(
        v16, idx,
        lax.GatherDimensionNumbers(offset_dims=(), collapsed_slice_dims=(0,),
                                   start_index_map=(0,)),
        (1,), mode=lax.GatherScatterMode.PROMISE_IN_BOUNDS)


CQ = 40
NQ = C // CQ


def _gcn_aggregate(support_flat, cols_p, rows_p, adj_p):
    mesh = plsc.VectorSubcoreMesh(core_axis_name="c", subcore_axis_name="s")

    @functools.partial(
        pl.kernel,
        out_type=jax.ShapeDtypeStruct((NCORES, SEQ, N, D), jnp.float32),
        mesh=mesh,
        scratch_types=[
            pltpu.VMEM((CQ, K), jnp.int32),
            pltpu.VMEM((CQ, K), jnp.int32),
            pltpu.VMEM((CQ, K), jnp.float32),
            pltpu.VMEM((K, D), jnp.float32),
            pltpu.VMEM((K, D), jnp.float32),
            pltpu.VMEM_SHARED((N, D), jnp.float32),
            pltpu.SemaphoreType.DMA,
            pltpu.SemaphoreType.DMA,
            pltpu.SemaphoreType.DMA,
            pltpu.SemaphoreType.DMA,
        ],
    )
    def k(sup_hbm, cols_hbm, rows_hbm, adj_hbm, out_hbm,
          cols_all, rows_all, adj_all, buf0, buf1, acc,
          semg0, semg1, sems0, sems1):
        cid = lax.axis_index("c")
        sid = lax.axis_index("s")
        w = cid * NSUB + sid
        base = sid * RPT
        last = sid == NSUB - 1

        def zero_acc():
            z16 = jnp.zeros((16,), jnp.float32)
            for v in range(D // 16):
                def zrow(r, _, v=v):
                    buf0[r, pl.ds(v * 16, 16)] = z16
                    return _
                lax.fori_loop(0, ZR, zrow, None)
            for z in range(RPT // ZR):
                pltpu.sync_copy(buf0.at[pl.ds(0, ZR)],
                                acc.at[pl.ds(base + z * ZR, ZR)])

            @pl.when(last)
            def _():
                pltpu.sync_copy(buf0.at[pl.ds(0, REXTRA)],
                                acc.at[pl.ds(NSUB * RPT, REXTRA)])

        def gath(m, buf, sem):
            return [pltpu.make_async_copy(
                        sup_hbm.at[cols_all.at[m]], buf, sem)]

        def scal(m, buf, sem):
            return pltpu.async_copy(buf, acc.at[rows_all.at[m]], sem,
                                    add=True)

        def scale(m, buf):
            def sg(g, _):
                a16 = adj_all[m, pl.ds(g * 16, 16)]
                for e in range(16):
                    jj = g * 16 + e
                    ae = _bcast_lane(a16, e)
                    for v in range(D // 16):
                        sl = pl.ds(v * 16, 16)
                        buf[jj, sl] = buf[jj, sl] * ae
                return _
            lax.fori_loop(0, K // 16, sg, None)

        zero_acc()
        plsc.subcore_barrier()

        for s in range(SEQ):
            for q in range(NQ):
                rbase = (s * NW + w) * C + q * CQ
                pltpu.sync_copy(cols_hbm.at[pl.ds(rbase, CQ)], cols_all)
                pltpu.sync_copy(rows_hbm.at[pl.ds(rbase, CQ)], rows_all)
                pltpu.sync_copy(adj_hbm.at[pl.ds(rbase, CQ)], adj_all)

                for d in gath(0, buf0, semg0):
                    d.start()
                for d in gath(1, buf1, semg1):
                    d.start()

                def body(j, _):
                    m0 = 2 * j
                    for d in gath(m0, buf0, semg0):
                        d.wait()
                    scale(m0, buf0)
                    sc0 = scal(m0, buf0, sems0)

                    for d in gath(m0 + 1, buf1, semg1):
                        d.wait()
                    scale(m0 + 1, buf1)
                    sc1 = scal(m0 + 1, buf1, sems1)

                    @pl.when(j < CQ // 2 - 1)
                    def _():
                        sc0.wait()
                        for d in gath(m0 + 2, buf0, semg0):
                            d.start()
                        sc1.wait()
                        for d in gath(m0 + 3, buf1, semg1):
                            d.start()

                    @pl.when(j == CQ // 2 - 1)
                    def _():
                        sc0.wait()
                        sc1.wait()
                    return _
                lax.fori_loop(0, CQ // 2, body, None)
            plsc.subcore_barrier()
            pltpu.sync_copy(acc.at[pl.ds(base, RPT)],
                            out_hbm.at[cid, s, pl.ds(base, RPT)])

            @pl.when(last)
            def _():
                pltpu.sync_copy(acc.at[pl.ds(NSUB * RPT, REXTRA)],
                                out_hbm.at[cid, s, pl.ds(NSUB * RPT, REXTRA)])

            if s < SEQ - 1:
                zero_acc()
            plsc.subcore_barrier()

    return k(support_flat, cols_p, rows_p, adj_p)


BT = 10


def _lstm_body(p0_ref, p1_ref, gb_ref, wih_ref, whh_ref, bias_ref,
               h0_ref, c0_ref, out_ref, h_scr, c_scr):
    t = pl.program_id(0)

    @pl.when(t == 0)
    def _():
        h_scr[...] = h0_ref[...]
        c_scr[...] = c0_ref[...]

    wih = wih_ref[...]
    whh = whh_ref[...]
    bias = bias_ref[...]
    gbv = gb_ref[...]

    def step(i, hc):
        h, c = hc
        rel = jnp.maximum(p0_ref[i] + p1_ref[i] + gbv, 0.0)
        gates = (jnp.dot(rel, wih, preferred_element_type=jnp.float32)
                 + jnp.dot(h, whh, preferred_element_type=jnp.float32)
                 + bias)
        i_g = jax.nn.sigmoid(gates[:, :H])
        f_g = jax.nn.sigmoid(gates[:, H:2 * H])
        g_g = jnp.tanh(gates[:, 2 * H:3 * H])
        o_g = jax.nn.sigmoid(gates[:, 3 * H:])
        c2 = f_g * c + i_g * g_g
        h2 = o_g * jnp.tanh(c2)
        out_ref[i] = h2
        return (h2, c2)

    h, c = lax.fori_loop(0, BT, step, (h_scr[...], c_scr[...]))
    h_scr[...] = h
    c_scr[...] = c


def _lstm(p0, p1, gb, wih_t, whh_t, bias, h0, c0):
    T = SEQ * N // B
    return pl.pallas_call(
        _lstm_body,
        grid=(T // BT,),
        in_specs=[
            pl.BlockSpec((BT, B, H), lambda i: (i, 0, 0)),
            pl.BlockSpec((BT, B, H), lambda i: (i, 0, 0)),
            pl.BlockSpec((1, H), lambda i: (0, 0)),
            pl.BlockSpec((D, 4 * H), lambda i: (0, 0)),
            pl.BlockSpec((H, 4 * H), lambda i: (0, 0)),
            pl.BlockSpec((1, 4 * H), lambda i: (0, 0)),
            pl.BlockSpec((B, H), lambda i: (0, 0)),
            pl.BlockSpec((B, H), lambda i: (0, 0)),
        ],
        out_specs=pl.BlockSpec((BT, B, H), lambda i: (i, 0, 0)),
        out_shape=jax.ShapeDtypeStruct((T, B, H), jnp.float32),
        scratch_shapes=[
            pltpu.VMEM((B, H), jnp.float32),
            pltpu.VMEM((B, H), jnp.float32),
        ],
    )(p0, p1, gb, wih_t, whh_t, bias, h0, c0)


def kernel(xs, edge_index, adj_values, gcn_w, gcn_b, W_ih, W_hh,
           b_ih, b_hh, h0, c0):
    xs_flat = xs.reshape(SEQ * N, D)
    support = _support_matmul(xs_flat, gcn_w)

    pad = NW * EPT - E
    cols = edge_index[:, 1, :] + (jnp.arange(SEQ, dtype=jnp.int32) * N)[:, None]
    rows = edge_index[:, 0, :]
    pad_idx = jnp.arange(pad, dtype=jnp.int32) % N
    pad_cols = pad_idx[None, :] + (jnp.arange(SEQ, dtype=jnp.int32) * N)[:, None]
    pad_rows = jnp.broadcast_to(pad_idx, (SEQ, pad))
    cols_p = jnp.concatenate([cols, pad_cols], axis=1).reshape(-1, K)
    rows_p = jnp.concatenate([rows, pad_rows], axis=1).reshape(-1, K)
    adj_p = jnp.concatenate(
        [adj_values, jnp.zeros((SEQ, pad), jnp.float32)], axis=1).reshape(-1, K)

    parts = _gcn_aggregate(support, cols_p, rows_p, adj_p)
    p0 = parts[0].reshape(SEQ * N // B, B, H)
    p1 = parts[1].reshape(SEQ * N // B, B, H)

    gb = gcn_b.reshape(1, H)
    wih_t = W_ih.T
    whh_t = W_hh.T
    bias = (b_ih + b_hh).reshape(1, 4 * H)

    return _lstm(p0, p1, gb, wih_t, whh_t, bias, h0[0], c0[0])
